# Initial kernel scaffold; baseline (speedup 1.0000x reference)
import jax
import jax.numpy as jnp
from jax.experimental import pallas as pl


def kernel(batch_walk, neg_idx_list_dst, node_embed_weight, context_embed_weight):
    raise NotImplementedError("write your pallas kernel here")


# speedup vs baseline: 7.9372x; 7.9372x over previous
import functools

import jax
import jax.numpy as jnp
from jax import lax
from jax.experimental import pallas as pl
from jax.experimental.pallas import tpu as pltpu
from jax.experimental.pallas import tpu_sc as plsc

D = 128
B = 512
WL = 40
WIN = 5
NEG = 5
T = B * WL
NNEG = T * NEG
NPOS = B * 2 * sum(WL - d for d in range(1, WIN + 1))

NC = 2
NS = 16
NW = NC * NS
TPW = T // NW
NEGPW = NNEG // NW
CH = 128
NCH_TOK = TPW // CH
NCH_NEG = NEGPW // CH


def _sc_gather_body(walk_hbm, negperm_hbm, node_hbm, ctx_hbm,
                    nodeg_hbm, ctxg_hbm, negctx_hbm,
                    walk_v, negidx_v, nid_v, rows_v, sem):
    wid = lax.axis_index("s") * NC + lax.axis_index("c")
    tbase = wid * TPW
    nbase = wid * NEGPW

    pltpu.sync_copy(walk_hbm.at[pl.ds(tbase, TPW)], walk_v)
    pltpu.sync_copy(negperm_hbm.at[pl.ds(nbase, NEGPW)], negidx_v)

    for c in range(NCH_TOK):
        idx = walk_v.at[pl.ds(c * CH, CH)]
        pltpu.async_copy(node_hbm.at[idx], rows_v, sem).wait()
        pltpu.sync_copy(rows_v, nodeg_hbm.at[pl.ds(tbase + c * CH, CH)])
        pltpu.async_copy(ctx_hbm.at[idx], rows_v, sem).wait()
        pltpu.sync_copy(rows_v, ctxg_hbm.at[pl.ds(tbase + c * CH, CH)])

    @pl.loop(0, NCH_NEG)
    def _neg_chunk(c):
        base = c * CH
        pltpu.async_copy(
            walk_hbm.at[negidx_v.at[pl.ds(base, CH)]], nid_v, sem).wait()
        pltpu.async_copy(ctx_hbm.at[nid_v], rows_v, sem).wait()
        pltpu.sync_copy(rows_v, negctx_hbm.at[pl.ds(nbase + base, CH)])


@functools.cache
def _sc_gather():
    return pl.kernel(
        _sc_gather_body,
        out_type=(
            jax.ShapeDtypeStruct((T, D), jnp.float32),
            jax.ShapeDtypeStruct((T, D), jnp.float32),
            jax.ShapeDtypeStruct((NNEG, D), jnp.float32),
        ),
        mesh=plsc.VectorSubcoreMesh(
            core_axis_name="c", subcore_axis_name="s",
            num_cores=NC, num_subcores=NS),
        scratch_types=(
            pltpu.VMEM((TPW,), jnp.int32),
            pltpu.VMEM((NEGPW,), jnp.int32),
            pltpu.VMEM((CH,), jnp.int32),
            pltpu.VMEM((CH, D), jnp.float32),
            pltpu.SemaphoreType.DMA,
        ),
    )


TC_GRID = 32
TB = T // TC_GRID


def _tc_loss_body(node_ref, ctx_ref, neg_ref, acc_ref):
    i = pl.program_id(0)
    node = node_ref[...]
    ctx = ctx_ref[...]
    rowpos = lax.broadcasted_iota(jnp.int32, (TB,), 0) % WL

    def nls(score):
        return jnp.log1p(jnp.exp(-jnp.clip(score, -6.0, 6.0)))

    pos_sum = jnp.float32(0.0)
    for d in range(1, WIN + 1):
        s = jnp.sum(node[: TB - d] * ctx[d:], axis=1)
        m = rowpos[: TB - d] < (WL - d)
        pos_sum += jnp.sum(jnp.where(m, nls(s), 0.0))
        s = jnp.sum(node[d:] * ctx[: TB - d], axis=1)
        m = rowpos[d:] >= d
        pos_sum += jnp.sum(jnp.where(m, nls(s), 0.0))

    neg_sum = jnp.float32(0.0)
    for k in range(NEG):
        s = jnp.sum(node * neg_ref[k], axis=1)
        neg_sum += jnp.sum(nls(-s))

    @pl.when(i == 0)
    def _init():
        acc_ref[0, 0] = jnp.float32(0.0)
        acc_ref[0, 1] = jnp.float32(0.0)

    acc_ref[0, 0] += pos_sum
    acc_ref[0, 1] += neg_sum


_tc_loss = pl.pallas_call(
    _tc_loss_body,
    grid=(TC_GRID,),
    in_specs=[
        pl.BlockSpec((TB, D), lambda i: (i, 0)),
        pl.BlockSpec((TB, D), lambda i: (i, 0)),
        pl.BlockSpec((NEG, TB, D), lambda i: (0, i, 0)),
    ],
    out_specs=pl.BlockSpec(memory_space=pltpu.SMEM),
    out_shape=jax.ShapeDtypeStruct((1, 2), jnp.float32),
)


def kernel(batch_walk, neg_idx_list_dst, node_embed_weight, context_embed_weight):
    flat_walk = batch_walk.reshape(-1)
    negperm = neg_idx_list_dst.reshape(T, NEG).T.reshape(-1)
    node_g, ctx_g, negctx = _sc_gather()(
        flat_walk, negperm, node_embed_weight, context_embed_weight)
    acc = _tc_loss(node_g, ctx_g, negctx.reshape(NEG, T, D))
    pos_loss = acc[0, 0] / NPOS
    neg_loss = acc[0, 1] * (NEG * 1.0) / NNEG
    return pos_loss + neg_loss


# speedup vs baseline: 11.2801x; 1.4212x over previous
import functools

import jax
import jax.numpy as jnp
from jax import lax
from jax.experimental import pallas as pl
from jax.experimental.pallas import tpu as pltpu
from jax.experimental.pallas import tpu_sc as plsc

D = 128
B = 512
WL = 40
WIN = 5
NEG = 5
T = B * WL
NNEG = T * NEG
NPOS = B * 2 * sum(WL - d for d in range(1, WIN + 1))

NC = 2
NS = 16
NW = NC * NS
TPW = T // NW
NEGPW = NNEG // NW
CH = 128
NCH_TOK = TPW // CH
NCH_NEG = NEGPW // CH


def _sc_gather_body(walk_hbm, negperm_hbm, node_hbm, ctx_hbm,
                    nodeg_hbm, ctxg_hbm, negctx_hbm,
                    walk_v, negidx_v, nid_v, rows_v, sem):
    wid = lax.axis_index("s") * NC + lax.axis_index("c")
    tbase = wid * TPW
    nbase = wid * NEGPW

    pltpu.sync_copy(walk_hbm.at[pl.ds(tbase, TPW)], walk_v)
    pltpu.sync_copy(negperm_hbm.at[pl.ds(nbase, NEGPW)], negidx_v)

    for c in range(NCH_TOK):
        idx = walk_v.at[pl.ds(c * CH, CH)]
        pltpu.async_copy(node_hbm.at[idx], rows_v, sem).wait()
        pltpu.sync_copy(rows_v, nodeg_hbm.at[pl.ds(tbase + c * CH, CH)])
        pltpu.async_copy(ctx_hbm.at[idx], rows_v, sem).wait()
        pltpu.sync_copy(rows_v, ctxg_hbm.at[pl.ds(tbase + c * CH, CH)])

    @pl.loop(0, NCH_NEG)
    def _neg_chunk(c):
        base = c * CH
        pltpu.async_copy(
            walk_hbm.at[negidx_v.at[pl.ds(base, CH)]], nid_v, sem).wait()
        pltpu.async_copy(ctx_hbm.at[nid_v], rows_v, sem).wait()
        pltpu.sync_copy(rows_v, negctx_hbm.at[pl.ds(nbase + base, CH)])


@functools.cache
def _sc_gather():
    return pl.kernel(
        _sc_gather_body,
        out_type=(
            jax.ShapeDtypeStruct((T, D), jnp.float32),
            jax.ShapeDtypeStruct((T, D), jnp.float32),
            jax.ShapeDtypeStruct((NNEG, D), jnp.float32),
        ),
        mesh=plsc.VectorSubcoreMesh(
            core_axis_name="c", subcore_axis_name="s",
            num_cores=NC, num_subcores=NS),
        scratch_types=(
            pltpu.VMEM((TPW,), jnp.int32),
            pltpu.VMEM((NEGPW,), jnp.int32),
            pltpu.VMEM((CH,), jnp.int32),
            pltpu.VMEM((CH, D), jnp.float32),
            pltpu.SemaphoreType.DMA,
        ),
    )


TC_GRID = 32
TB = T // TC_GRID


WPB = TB // WL


def _tc_loss_body(node_ref, ctx_ref, neg_ref, acc_ref):
    i = pl.program_id(0)
    node = node_ref[...]
    ctx = ctx_ref[...]

    def nls(score):
        return jnp.log1p(jnp.exp(-jnp.clip(score, -6.0, 6.0)))

    nb = node.astype(jnp.bfloat16)
    cb = ctx.astype(jnp.bfloat16)
    jj = lax.broadcasted_iota(jnp.int32, (WL, WL), 0)
    ii = lax.broadcasted_iota(jnp.int32, (WL, WL), 1)
    dd = ii - jj
    band = (dd != 0) & (dd >= -WIN) & (dd <= WIN)
    pos_sum = jnp.float32(0.0)
    for w in range(WPB):
        a = nb[w * WL:(w + 1) * WL]
        b = cb[w * WL:(w + 1) * WL]
        g = lax.dot_general(a, b, (((1,), (1,)), ((), ())),
                            preferred_element_type=jnp.float32)
        pos_sum += jnp.sum(jnp.where(band, nls(g), 0.0))

    neg_sum = jnp.float32(0.0)
    for k in range(NEG):
        s = jnp.sum(node * neg_ref[k], axis=1)
        neg_sum += jnp.sum(nls(-s))

    @pl.when(i == 0)
    def _init():
        acc_ref[0, 0] = jnp.float32(0.0)
        acc_ref[0, 1] = jnp.float32(0.0)

    acc_ref[0, 0] += pos_sum
    acc_ref[0, 1] += neg_sum


_tc_loss = pl.pallas_call(
    _tc_loss_body,
    grid=(TC_GRID,),
    in_specs=[
        pl.BlockSpec((TB, D), lambda i: (i, 0)),
        pl.BlockSpec((TB, D), lambda i: (i, 0)),
        pl.BlockSpec((NEG, TB, D), lambda i: (0, i, 0)),
    ],
    out_specs=pl.BlockSpec(memory_space=pltpu.SMEM),
    out_shape=jax.ShapeDtypeStruct((1, 2), jnp.float32),
)


def kernel(batch_walk, neg_idx_list_dst, node_embed_weight, context_embed_weight):
    flat_walk = batch_walk.reshape(-1)
    negperm = neg_idx_list_dst.reshape(T, NEG).T.reshape(-1)
    node_g, ctx_g, negctx = _sc_gather()(
        flat_walk, negperm, node_embed_weight, context_embed_weight)
    acc = _tc_loss(node_g, ctx_g, negctx.reshape(NEG, T, D))
    pos_loss = acc[0, 0] / NPOS
    neg_loss = acc[0, 1] * (NEG * 1.0) / NNEG
    return pos_loss + neg_loss


# speedup vs baseline: 14.0532x; 1.2458x over previous
import functools

import jax
import jax.numpy as jnp
from jax import lax
from jax.experimental import pallas as pl
from jax.experimental.pallas import tpu as pltpu
from jax.experimental.pallas import tpu_sc as plsc

D = 128
B = 512
WL = 40
WIN = 5
NEG = 5
T = B * WL
NNEG = T * NEG
NPOS = B * 2 * sum(WL - d for d in range(1, WIN + 1))

NC = 2
NS = 16
NW = NC * NS
TPW = T // NW
NEGPW = NNEG // NW
CH = 128
NCH_TOK = TPW // CH
NCH_NEG = NEGPW // CH


NRING = 4


def _sc_gather_body(walk_hbm, negperm_hbm, node_hbm, ctx_hbm,
                    nodeg_hbm, ctxg_hbm, negctx_hbm,
                    walk_v, negidx_v, nid_v, rows_v,
                    csem, gs0, gs1, gs2, gs3, os0, os1, os2, os3):
    gsems = (gs0, gs1, gs2, gs3)
    osems = (os0, os1, os2, os3)
    wid = lax.axis_index("s") * NC + lax.axis_index("c")
    tbase = wid * TPW
    nbase = wid * NEGPW

    pltpu.sync_copy(walk_hbm.at[pl.ds(tbase, TPW)], walk_v)

    def tok_gather(j, b):
        idx = walk_v.at[pl.ds((j // 2) * CH, CH)]
        tab = node_hbm if j % 2 == 0 else ctx_hbm
        pltpu.async_copy(tab.at[idx], rows_v.at[b], gsems[b])

    def tok_out(j, b):
        dst = nodeg_hbm if j % 2 == 0 else ctxg_hbm
        pltpu.async_copy(
            rows_v.at[b], dst.at[pl.ds(tbase + (j // 2) * CH, CH)], osems[b])

    def neg_gather(c, b):
        pltpu.async_copy(
            ctx_hbm.at[nid_v.at[pl.ds(c * CH, CH)]], rows_v.at[b], gsems[b])

    def neg_out(c, b):
        pltpu.async_copy(
            rows_v.at[b], negctx_hbm.at[pl.ds(nbase + c * CH, CH)], osems[b])

    def wait_g(b):
        pltpu.make_async_copy(
            node_hbm.at[pl.ds(0, CH)], rows_v.at[b], gsems[b]).wait()

    def wait_o(b):
        pltpu.make_async_copy(
            rows_v.at[b], nodeg_hbm.at[pl.ds(0, CH)], osems[b]).wait()

    for j in range(NRING):
        tok_gather(j, j)
    pltpu.sync_copy(negperm_hbm.at[pl.ds(nbase, NEGPW)], negidx_v)

    @pl.loop(0, NCH_NEG)
    def _compose(c):
        pltpu.async_copy(
            walk_hbm.at[negidx_v.at[pl.ds(c * CH, CH)]],
            nid_v.at[pl.ds(c * CH, CH)], csem)

    pltpu.make_async_copy(
        walk_hbm.at[pl.ds(0, NEGPW)], nid_v, csem).wait()

    for j in range(2 * NCH_TOK):
        b = j % NRING
        wait_g(b)
        tok_out(j, b)
        wait_o(b)
        nxt = j + NRING
        if nxt < 2 * NCH_TOK:
            tok_gather(nxt, b)
        else:
            neg_gather(nxt - 2 * NCH_TOK, b)

    @pl.loop(0, (NCH_NEG - 1) // NRING)
    def _neg_group(g):
        for i in range(NRING):
            c = g * NRING + i
            wait_g(i)
            neg_out(c, i)
            wait_o(i)
            nc = c + NRING

            @pl.when(nc < NCH_NEG)
            def _():
                neg_gather(nc, i)

    b = (NCH_NEG - 1) % NRING
    wait_g(b)
    neg_out(NCH_NEG - 1, b)
    wait_o(b)


@functools.cache
def _sc_gather():
    return pl.kernel(
        _sc_gather_body,
        out_type=(
            jax.ShapeDtypeStruct((T, D), jnp.float32),
            jax.ShapeDtypeStruct((T, D), jnp.float32),
            jax.ShapeDtypeStruct((NNEG, D), jnp.float32),
        ),
        mesh=plsc.VectorSubcoreMesh(
            core_axis_name="c", subcore_axis_name="s",
            num_cores=NC, num_subcores=NS),
        scratch_types=(
            pltpu.VMEM((TPW,), jnp.int32),
            pltpu.VMEM((NEGPW,), jnp.int32),
            pltpu.VMEM((NEGPW,), jnp.int32),
            pltpu.VMEM((NRING, CH, D), jnp.float32),
        ) + (pltpu.SemaphoreType.DMA,) * 9,
    )


TC_GRID = 32
TB = T // TC_GRID


WPB = TB // WL


def _tc_loss_body(node_ref, ctx_ref, neg_ref, acc_ref):
    i = pl.program_id(0)
    node = node_ref[...]
    ctx = ctx_ref[...]

    def nls(score):
        return jnp.log1p(jnp.exp(-jnp.clip(score, -6.0, 6.0)))

    nb = node.astype(jnp.bfloat16)
    cb = ctx.astype(jnp.bfloat16)
    jj = lax.broadcasted_iota(jnp.int32, (WL, WL), 0)
    ii = lax.broadcasted_iota(jnp.int32, (WL, WL), 1)
    dd = ii - jj
    band = (dd != 0) & (dd >= -WIN) & (dd <= WIN)
    pos_sum = jnp.float32(0.0)
    for w in range(WPB):
        a = nb[w * WL:(w + 1) * WL]
        b = cb[w * WL:(w + 1) * WL]
        g = lax.dot_general(a, b, (((1,), (1,)), ((), ())),
                            preferred_element_type=jnp.float32)
        pos_sum += jnp.sum(jnp.where(band, nls(g), 0.0))

    neg_sum = jnp.float32(0.0)
    for k in range(NEG):
        s = jnp.sum(node * neg_ref[k], axis=1)
        neg_sum += jnp.sum(nls(-s))

    @pl.when(i == 0)
    def _init():
        acc_ref[0, 0] = jnp.float32(0.0)
        acc_ref[0, 1] = jnp.float32(0.0)

    acc_ref[0, 0] += pos_sum
    acc_ref[0, 1] += neg_sum


_tc_loss = pl.pallas_call(
    _tc_loss_body,
    grid=(TC_GRID,),
    in_specs=[
        pl.BlockSpec((TB, D), lambda i: (i, 0)),
        pl.BlockSpec((TB, D), lambda i: (i, 0)),
        pl.BlockSpec((NEG, TB, D), lambda i: (0, i, 0)),
    ],
    out_specs=pl.BlockSpec(memory_space=pltpu.SMEM),
    out_shape=jax.ShapeDtypeStruct((1, 2), jnp.float32),
)


def kernel(batch_walk, neg_idx_list_dst, node_embed_weight, context_embed_weight):
    flat_walk = batch_walk.reshape(-1)
    negperm = neg_idx_list_dst.reshape(T, NEG).T.reshape(-1)
    node_g, ctx_g, negctx = _sc_gather()(
        flat_walk, negperm, node_embed_weight, context_embed_weight)
    acc = _tc_loss(node_g, ctx_g, negctx.reshape(NEG, T, D))
    pos_loss = acc[0, 0] / NPOS
    neg_loss = acc[0, 1] * (NEG * 1.0) / NNEG
    return pos_loss + neg_loss


# speedup vs baseline: 18.6923x; 1.3301x over previous
import functools

import jax
import jax.numpy as jnp
from jax import lax
from jax.experimental import pallas as pl
from jax.experimental.pallas import tpu as pltpu
from jax.experimental.pallas import tpu_sc as plsc

D = 128
B = 512
WL = 40
WIN = 5
NEG = 5
T = B * WL
NNEG = T * NEG
NPOS = B * 2 * sum(WL - d for d in range(1, WIN + 1))

NC = 2
NS = 16
NW = NC * NS
TPW = T // NW
NEGPW = NNEG // NW
CH = 128
NCH_TOK = TPW // CH
CCH = 64
NCH_CTX = TPW // CCH

NTOK_CH = 16
NEG_CH = NTOK_CH * NEG
NCH_NEG = TPW // NTOK_CH
PPR = NEG_CH // 8
PPG = 4
PP_ROWS = NNEG // 8


def _sc_gather_body(walk_hbm, negidx_hbm, node_hbm, ctx_hbm,
                    nodeg_hbm, ctxg_hbm, pp_hbm,
                    walk_v, negidx_v, nid_v, node_v, crow_v, nrow_v, pp_v,
                    nsem, csem, wsem, psem, gs0, gs1, os0, os1, ng0, ng1):
    gsems = (gs0, gs1)
    osems = (os0, os1)
    ngsems = (ng0, ng1)
    wid = lax.axis_index("s") * NC + lax.axis_index("c")
    tbase = wid * TPW
    nbase = wid * NEGPW
    pbase = wid * (NEGPW // 8)

    pltpu.sync_copy(walk_hbm.at[pl.ds(tbase, TPW)], walk_v)

    for c in range(NCH_TOK):
        pltpu.async_copy(node_hbm.at[walk_v.at[pl.ds(c * CH, CH)]],
                         node_v.at[pl.ds(c * CH, CH)], nsem)

    pltpu.sync_copy(negidx_hbm.at[pl.ds(nbase, NEGPW)], negidx_v)

    @pl.loop(0, NEGPW // CH)
    def _compose(c):
        pltpu.async_copy(walk_hbm.at[negidx_v.at[pl.ds(c * CH, CH)]],
                         nid_v.at[pl.ds(c * CH, CH)], csem)

    def ctx_gather(j, b):
        pltpu.async_copy(ctx_hbm.at[walk_v.at[pl.ds(j * CCH, CCH)]],
                         crow_v.at[b], gsems[b])

    for j in range(2):
        ctx_gather(j, j)
    for j in range(NCH_CTX):
        b = j % 2
        pltpu.make_async_copy(
            ctx_hbm.at[pl.ds(0, CCH)], crow_v.at[b], gsems[b]).wait()
        pltpu.async_copy(
            crow_v.at[b], ctxg_hbm.at[pl.ds(tbase + j * CCH, CCH)], osems[b])
        pltpu.make_async_copy(
            crow_v.at[b], ctxg_hbm.at[pl.ds(0, CCH)], osems[b]).wait()
        if j + 2 < NCH_CTX:
            ctx_gather(j + 2, b)

    pltpu.make_async_copy(
        node_hbm.at[pl.ds(0, TPW)], node_v, nsem).wait()
    pltpu.async_copy(node_v, nodeg_hbm.at[pl.ds(tbase, TPW)], wsem)

    pltpu.make_async_copy(
        walk_hbm.at[pl.ds(0, NEGPW)], nid_v, csem).wait()

    def neg_gather(c, b):
        pltpu.async_copy(
            ctx_hbm.at[nid_v.at[pl.ds(c * NEG_CH, NEG_CH)]],
            nrow_v.at[b], ngsems[b])

    for c in range(2):
        neg_gather(c, c)

    @pl.loop(0, NCH_NEG // PPG)
    def _neg_group(g):
        for cc in range(PPG):
            c = g * PPG + cc
            rb = cc % 2
            pltpu.make_async_copy(
                ctx_hbm.at[pl.ds(0, NEG_CH)], nrow_v.at[rb], ngsems[rb]).wait()

            @pl.loop(0, NTOK_CH)
            def _tok(tt):
                trow = c * NTOK_CH + tt
                nd = [node_v[trow, pl.ds(q * 16, 16)] for q in range(8)]
                for k in range(NEG):
                    r = tt * NEG + k
                    acc0 = nd[0] * nrow_v[rb, r, pl.ds(0, 16)]
                    acc1 = nd[1] * nrow_v[rb, r, pl.ds(16, 16)]
                    for q in range(2, 8, 2):
                        acc0 += nd[q] * nrow_v[rb, r, pl.ds(q * 16, 16)]
                        acc1 += nd[q + 1] * nrow_v[rb, r, pl.ds(q * 16 + 16, 16)]
                    pp_v[cc * PPR + lax.shift_right_logical(r, 3),
                         pl.ds(lax.shift_left(lax.bitwise_and(r, 7), 4), 16)] = (
                        acc0 + acc1)

            nc = c + 2

            @pl.when(nc < NCH_NEG)
            def _():
                neg_gather(nc, rb)

        pltpu.async_copy(
            pp_v, pp_hbm.at[pl.ds(pbase + g * (PPG * PPR), PPG * PPR)], psem)
        pltpu.make_async_copy(
            pp_v, pp_hbm.at[pl.ds(0, PPG * PPR)], psem).wait()

    pltpu.make_async_copy(node_v, nodeg_hbm.at[pl.ds(0, TPW)], wsem).wait()


@functools.cache
def _sc_gather():
    return pl.kernel(
        _sc_gather_body,
        out_type=(
            jax.ShapeDtypeStruct((T, D), jnp.float32),
            jax.ShapeDtypeStruct((T, D), jnp.float32),
            jax.ShapeDtypeStruct((PP_ROWS, D), jnp.float32),
        ),
        mesh=plsc.VectorSubcoreMesh(
            core_axis_name="c", subcore_axis_name="s",
            num_cores=NC, num_subcores=NS),
        scratch_types=(
            pltpu.VMEM((TPW,), jnp.int32),
            pltpu.VMEM((NEGPW,), jnp.int32),
            pltpu.VMEM((NEGPW,), jnp.int32),
            pltpu.VMEM((TPW, D), jnp.float32),
            pltpu.VMEM((2, CCH, D), jnp.float32),
            pltpu.VMEM((2, NEG_CH, D), jnp.float32),
            pltpu.VMEM((PPG * PPR, D), jnp.float32),
        ) + (pltpu.SemaphoreType.DMA,) * 10,
    )


TC_GRID = 32
TB = T // TC_GRID
WPB = TB // WL
PPB = PP_ROWS // TC_GRID


def _tc_loss_body(node_ref, ctx_ref, pp_ref, acc_ref):
    i = pl.program_id(0)
    node = node_ref[...]
    ctx = ctx_ref[...]

    def nls(score):
        return jnp.log1p(jnp.exp(-jnp.clip(score, -6.0, 6.0)))

    nb = node.astype(jnp.bfloat16)
    cb = ctx.astype(jnp.bfloat16)
    jj = lax.broadcasted_iota(jnp.int32, (WL, WL), 0)
    ii = lax.broadcasted_iota(jnp.int32, (WL, WL), 1)
    dd = ii - jj
    band = (dd != 0) & (dd >= -WIN) & (dd <= WIN)
    pos_sum = jnp.float32(0.0)
    for w in range(WPB):
        a = nb[w * WL:(w + 1) * WL]
        b = cb[w * WL:(w + 1) * WL]
        g = lax.dot_general(a, b, (((1,), (1,)), ((), ())),
                            preferred_element_type=jnp.float32)
        pos_sum += jnp.sum(jnp.where(band, nls(g), 0.0))

    seg = (lax.broadcasted_iota(jnp.int32, (D, 8), 0) // 16
           == lax.broadcasted_iota(jnp.int32, (D, 8), 1)).astype(jnp.float32)
    s8 = lax.dot_general(pp_ref[...], seg, (((1,), (0,)), ((), ())),
                         preferred_element_type=jnp.float32)
    neg_sum = jnp.sum(nls(-s8))

    @pl.when(i == 0)
    def _init():
        acc_ref[0, 0] = jnp.float32(0.0)
        acc_ref[0, 1] = jnp.float32(0.0)

    acc_ref[0, 0] += pos_sum
    acc_ref[0, 1] += neg_sum


_tc_loss = pl.pallas_call(
    _tc_loss_body,
    grid=(TC_GRID,),
    in_specs=[
        pl.BlockSpec((TB, D), lambda i: (i, 0)),
        pl.BlockSpec((TB, D), lambda i: (i, 0)),
        pl.BlockSpec((PPB, D), lambda i: (i, 0)),
    ],
    out_specs=pl.BlockSpec(memory_space=pltpu.SMEM),
    out_shape=jax.ShapeDtypeStruct((1, 2), jnp.float32),
)


def kernel(batch_walk, neg_idx_list_dst, node_embed_weight, context_embed_weight):
    flat_walk = batch_walk.reshape(-1)
    node_g, ctx_g, pp = _sc_gather()(
        flat_walk, neg_idx_list_dst, node_embed_weight, context_embed_weight)
    acc = _tc_loss(node_g, ctx_g, pp)
    pos_loss = acc[0, 0] / NPOS
    neg_loss = acc[0, 1] * (NEG * 1.0) / NNEG
    return pos_loss + neg_loss


# speedup vs baseline: 20.4429x; 1.0937x over previous
import functools

import jax
import jax.numpy as jnp
from jax import lax
from jax.experimental import pallas as pl
from jax.experimental.pallas import tpu as pltpu
from jax.experimental.pallas import tpu_sc as plsc

D = 128
B = 512
WL = 40
WIN = 5
NEG = 5
T = B * WL
NNEG = T * NEG
NPOS = B * 2 * sum(WL - d for d in range(1, WIN + 1))

NC = 2
NS = 16
NW = NC * NS
TPW = T // NW
NEGPW = NNEG // NW
CH = 128
NCH_TOK = TPW // CH

NTOK_CH = 16
NEG_CH = NTOK_CH * NEG
NCH_NEG = TPW // NTOK_CH
PPR = NEG_CH // 8
PPG = 4
PP_ROWS = NNEG // 8

_SC_MESH = dict(core_axis_name="c", subcore_axis_name="s",
                num_cores=NC, num_subcores=NS)


def _worker_id():
    return lax.axis_index("s") * NC + lax.axis_index("c")


def _sc_tok_body(walk_hbm, node_hbm, ctx_hbm, nodeg_hbm, ctxg_hbm,
                 walk_v, rows_v, gs0, gs1, gs2, gs3, os0, os1, os2, os3):
    gsems = (gs0, gs1, gs2, gs3)
    osems = (os0, os1, os2, os3)
    wid = _worker_id()
    tbase = wid * TPW

    pltpu.sync_copy(walk_hbm.at[pl.ds(tbase, TPW)], walk_v)

    def gather(j, b):
        idx = walk_v.at[pl.ds((j // 2) * CH, CH)]
        tab = node_hbm if j % 2 == 0 else ctx_hbm
        pltpu.async_copy(tab.at[idx], rows_v.at[b], gsems[b])

    def out(j, b):
        dst = nodeg_hbm if j % 2 == 0 else ctxg_hbm
        pltpu.async_copy(
            rows_v.at[b], dst.at[pl.ds(tbase + (j // 2) * CH, CH)], osems[b])

    for j in range(4):
        gather(j, j)
    for j in range(2 * NCH_TOK):
        b = j % 4
        pltpu.make_async_copy(
            node_hbm.at[pl.ds(0, CH)], rows_v.at[b], gsems[b]).wait()
        out(j, b)
        pltpu.make_async_copy(
            rows_v.at[b], nodeg_hbm.at[pl.ds(0, CH)], osems[b]).wait()
        if j + 4 < 2 * NCH_TOK:
            gather(j + 4, b)


@functools.cache
def _sc_tok():
    return pl.kernel(
        _sc_tok_body,
        out_type=(
            jax.ShapeDtypeStruct((T, D), jnp.float32),
            jax.ShapeDtypeStruct((T, D), jnp.float32),
        ),
        mesh=plsc.VectorSubcoreMesh(**_SC_MESH),
        scratch_types=(
            pltpu.VMEM((TPW,), jnp.int32),
            pltpu.VMEM((4, CH, D), jnp.float32),
        ) + (pltpu.SemaphoreType.DMA,) * 8,
    )


def _sc_neg_body(walk_hbm, negidx_hbm, ctx_hbm, nodeg_hbm, pp_hbm,
                 negidx_v, nid_v, node_v, nrow_v, pp_v,
                 nsem, csem, psem, ng0, ng1):
    ngsems = (ng0, ng1)
    wid = _worker_id()
    tbase = wid * TPW
    nbase = wid * NEGPW
    pbase = wid * (NEGPW // 8)

    pltpu.async_copy(nodeg_hbm.at[pl.ds(tbase, TPW)], node_v, nsem)

    pltpu.sync_copy(negidx_hbm.at[pl.ds(nbase, NEGPW)], negidx_v)

    @pl.loop(0, NEGPW // CH)
    def _compose(c):
        pltpu.async_copy(walk_hbm.at[negidx_v.at[pl.ds(c * CH, CH)]],
                         nid_v.at[pl.ds(c * CH, CH)], csem)

    pltpu.make_async_copy(
        walk_hbm.at[pl.ds(0, NEGPW)], nid_v, csem).wait()

    def neg_gather(c, b):
        pltpu.async_copy(
            ctx_hbm.at[nid_v.at[pl.ds(c * NEG_CH, NEG_CH)]],
            nrow_v.at[b], ngsems[b])

    for c in range(2):
        neg_gather(c, c)

    pltpu.make_async_copy(
        nodeg_hbm.at[pl.ds(0, TPW)], node_v, nsem).wait()

    @pl.loop(0, NCH_NEG // PPG)
    def _neg_group(g):
        for cc in range(PPG):
            c = g * PPG + cc
            rb = cc % 2
            pltpu.make_async_copy(
                ctx_hbm.at[pl.ds(0, NEG_CH)], nrow_v.at[rb], ngsems[rb]).wait()

            @pl.loop(0, NTOK_CH)
            def _tok(tt):
                trow = c * NTOK_CH + tt
                nd = [node_v[trow, pl.ds(q * 16, 16)] for q in range(8)]
                for k in range(NEG):
                    r = tt * NEG + k
                    acc0 = nd[0] * nrow_v[rb, r, pl.ds(0, 16)]
                    acc1 = nd[1] * nrow_v[rb, r, pl.ds(16, 16)]
                    for q in range(2, 8, 2):
                        acc0 += nd[q] * nrow_v[rb, r, pl.ds(q * 16, 16)]
                        acc1 += nd[q + 1] * nrow_v[rb, r, pl.ds(q * 16 + 16, 16)]
                    pp_v[cc * PPR + lax.shift_right_logical(r, 3),
                         pl.ds(lax.shift_left(lax.bitwise_and(r, 7), 4), 16)] = (
                        acc0 + acc1)

            nc = c + 2

            @pl.when(nc < NCH_NEG)
            def _():
                neg_gather(nc, rb)

        pltpu.async_copy(
            pp_v, pp_hbm.at[pl.ds(pbase + g * (PPG * PPR), PPG * PPR)], psem)
        pltpu.make_async_copy(
            pp_v, pp_hbm.at[pl.ds(0, PPG * PPR)], psem).wait()


@functools.cache
def _sc_neg():
    return pl.kernel(
        _sc_neg_body,
        out_type=jax.ShapeDtypeStruct((PP_ROWS, D), jnp.float32),
        mesh=plsc.VectorSubcoreMesh(**_SC_MESH),
        scratch_types=(
            pltpu.VMEM((NEGPW,), jnp.int32),
            pltpu.VMEM((NEGPW,), jnp.int32),
            pltpu.VMEM((TPW, D), jnp.float32),
            pltpu.VMEM((2, NEG_CH, D), jnp.float32),
            pltpu.VMEM((PPG * PPR, D), jnp.float32),
        ) + (pltpu.SemaphoreType.DMA,) * 5,
    )


TC_GRID = 32
TB = T // TC_GRID
WPB = TB // WL
NEG_GRID = 8
PPB = PP_ROWS // NEG_GRID


def _nls(score):
    return jnp.log1p(jnp.exp(-jnp.clip(score, -6.0, 6.0)))


def _tc_pos_body(node_ref, ctx_ref, acc_ref):
    i = pl.program_id(0)
    nb = node_ref[...].astype(jnp.bfloat16)
    cb = ctx_ref[...].astype(jnp.bfloat16)
    jj = lax.broadcasted_iota(jnp.int32, (WL, WL), 0)
    ii = lax.broadcasted_iota(jnp.int32, (WL, WL), 1)
    dd = ii - jj
    band = (dd != 0) & (dd >= -WIN) & (dd <= WIN)
    pos_sum = jnp.float32(0.0)
    for w in range(WPB):
        a = nb[w * WL:(w + 1) * WL]
        b = cb[w * WL:(w + 1) * WL]
        g = lax.dot_general(a, b, (((1,), (1,)), ((), ())),
                            preferred_element_type=jnp.float32)
        pos_sum += jnp.sum(jnp.where(band, _nls(g), 0.0))

    @pl.when(i == 0)
    def _init():
        acc_ref[0, 0] = jnp.float32(0.0)

    acc_ref[0, 0] += pos_sum


_tc_pos = pl.pallas_call(
    _tc_pos_body,
    grid=(TC_GRID,),
    in_specs=[
        pl.BlockSpec((TB, D), lambda i: (i, 0)),
        pl.BlockSpec((TB, D), lambda i: (i, 0)),
    ],
    out_specs=pl.BlockSpec(memory_space=pltpu.SMEM),
    out_shape=jax.ShapeDtypeStruct((1, 1), jnp.float32),
)


def _tc_neg_body(pp_ref, acc_ref):
    i = pl.program_id(0)
    seg = (lax.broadcasted_iota(jnp.int32, (D, 8), 0) // 16
           == lax.broadcasted_iota(jnp.int32, (D, 8), 1)).astype(jnp.float32)
    s8 = lax.dot_general(pp_ref[...], seg, (((1,), (0,)), ((), ())),
                         preferred_element_type=jnp.float32)
    neg_sum = jnp.sum(_nls(-s8))

    @pl.when(i == 0)
    def _init():
        acc_ref[0, 0] = jnp.float32(0.0)

    acc_ref[0, 0] += neg_sum


_tc_neg = pl.pallas_call(
    _tc_neg_body,
    grid=(NEG_GRID,),
    in_specs=[pl.BlockSpec((PPB, D), lambda i: (i, 0))],
    out_specs=pl.BlockSpec(memory_space=pltpu.SMEM),
    out_shape=jax.ShapeDtypeStruct((1, 1), jnp.float32),
)


def kernel(batch_walk, neg_idx_list_dst, node_embed_weight, context_embed_weight):
    flat_walk = batch_walk.reshape(-1)
    node_g, ctx_g = _sc_tok()(flat_walk, node_embed_weight, context_embed_weight)
    pp = _sc_neg()(flat_walk, neg_idx_list_dst, context_embed_weight, node_g)
    pos_acc = _tc_pos(node_g, ctx_g)
    neg_acc = _tc_neg(pp)
    pos_loss = pos_acc[0, 0] / NPOS
    neg_loss = neg_acc[0, 0] * (NEG * 1.0) / NNEG
    return pos_loss + neg_loss
